# baseline (speedup 1.0000x reference)
import functools
import jax
import jax.numpy as jnp
from jax import lax
from jax.experimental import pallas as pl
from jax.experimental.pallas import tpu as pltpu
from jax.experimental.pallas import tpu_sc as plsc

N = 10000
C = 128
P = 12
F = C * P
E = 320000
EROWS = 2560
EB = 125
NC = 2
NS = 16
ROWS_PER_TILE_DEG = EROWS // (NC * NS)
ROWS_PER_TILE_AGG = EROWS // NS
CP = 48
NPASS = F // (NC * CP)
NSLICE = N // 10

_sc_mesh = plsc.VectorSubcoreMesh(
    core_axis_name="c", subcore_axis_name="s", num_cores=NC, num_subcores=NS)


def _deg_body(dst_hbm, out_hbm, dst_v, ones_v, zeros_v, stage_v, acc_sh, sem):
    cid = lax.axis_index("c")
    sid = lax.axis_index("s")
    for i in range(63):
        zeros_v[pl.ds(i * 16, 16)] = jnp.zeros((16,), jnp.float32)
    for i in range(8):
        ones_v[pl.ds(i * 16, 16)] = jnp.ones((16,), jnp.float32)

    @pl.when(sid < 10)
    def _():
        pltpu.sync_copy(zeros_v.at[pl.ds(0, NSLICE)],
                        acc_sh.at[pl.ds(sid * NSLICE, NSLICE)])

    plsc.subcore_barrier()

    row0 = (cid * NS + sid) * ROWS_PER_TILE_DEG
    pltpu.sync_copy(dst_hbm.at[pl.ds(row0, ROWS_PER_TILE_DEG)], dst_v)

    def body(j, carry):
        pltpu.sync_copy(ones_v.at[pl.ds(0, EB)], acc_sh.at[dst_v.at[j]],
                        add=True)
        return carry

    lax.fori_loop(0, ROWS_PER_TILE_DEG, body, 0)
    plsc.subcore_barrier()

    @pl.when(sid < 10)
    def _():
        pltpu.sync_copy(acc_sh.at[pl.ds(sid * NSLICE, NSLICE)], stage_v)
        pltpu.sync_copy(stage_v,
                        out_hbm.at[pl.ds(cid * N + sid * NSLICE, NSLICE)])


_deg_call = pl.kernel(
    _deg_body,
    out_type=jax.ShapeDtypeStruct((NC * N,), jnp.float32),
    mesh=_sc_mesh,
    scratch_types=[
        pltpu.VMEM((ROWS_PER_TILE_DEG, EB), jnp.int32),
        pltpu.VMEM((128,), jnp.float32),
        pltpu.VMEM((1008,), jnp.float32),
        pltpu.VMEM((NSLICE,), jnp.float32),
        pltpu.VMEM_SHARED((N,), jnp.float32),
        pltpu.SemaphoreType.DMA,
    ],
)


NPAIR = ROWS_PER_TILE_AGG // 2
STROWS = N // 8 // 5


def _agg_body(src_hbm, dst_hbm, y_hbm, out_hbm, src_v, dst_v, buf0, buf1,
              st0, st1, table_sh, acc_sh, g0, g1, s0, s1):
    cid = lax.axis_index("c")
    sid = lax.axis_index("s")
    row0 = sid * ROWS_PER_TILE_AGG
    pltpu.sync_copy(src_hbm.at[pl.ds(row0, ROWS_PER_TILE_AGG)], src_v)
    pltpu.sync_copy(dst_hbm.at[pl.ds(row0, ROWS_PER_TILE_AGG)], dst_v)

    def one_pass(p, pcarry):
        sl = cid * NPASS + p

        @pl.when(sid < 10)
        def _():
            def stage_in(k, carry):
                r = sid * NSLICE + k * STROWS
                pltpu.sync_copy(y_hbm.at[sl, pl.ds(r, STROWS), :], st0)
                pltpu.sync_copy(st0, table_sh.at[pl.ds(r, STROWS), :])
                pltpu.sync_copy(st0, acc_sh.at[pl.ds(r, STROWS), :])
                return carry

            lax.fori_loop(0, NSLICE // STROWS, stage_in, 0)

        plsc.subcore_barrier()

        def body(i, carry):
            r = i * 2
            d0 = pltpu.async_copy(table_sh.at[src_v.at[r]], buf0, g0)
            d1 = pltpu.async_copy(table_sh.at[src_v.at[r + 1]], buf1, g1)
            d0.wait()
            pltpu.sync_copy(buf0, acc_sh.at[dst_v.at[r]], add=True)
            d1.wait()
            pltpu.sync_copy(buf1, acc_sh.at[dst_v.at[r + 1]], add=True)
            return carry

        lax.fori_loop(0, NPAIR, body, 0)
        plsc.subcore_barrier()

        @pl.when(sid < 10)
        def _():
            def stage_out(k, carry):
                r = sid * NSLICE + k * STROWS
                pltpu.sync_copy(acc_sh.at[pl.ds(r, STROWS), :], st0)
                pltpu.sync_copy(st0, out_hbm.at[sl, pl.ds(r, STROWS), :])
                return carry

            lax.fori_loop(0, NSLICE // STROWS, stage_out, 0)

        return pcarry

    lax.fori_loop(0, NPASS, one_pass, 0)


_agg_call = pl.kernel(
    _agg_body,
    out_type=jax.ShapeDtypeStruct((F // CP, N, CP), jnp.float32),
    mesh=_sc_mesh,
    scratch_types=[
        pltpu.VMEM((ROWS_PER_TILE_AGG, EB), jnp.int32),
        pltpu.VMEM((ROWS_PER_TILE_AGG, EB), jnp.int32),
        pltpu.VMEM((EB, CP), jnp.float32),
        pltpu.VMEM((EB, CP), jnp.float32),
        pltpu.VMEM((STROWS, CP), jnp.float32),
        pltpu.VMEM((STROWS, CP), jnp.float32),
        pltpu.VMEM_SHARED((N, CP), jnp.float32),
        pltpu.VMEM_SHARED((N, CP), jnp.float32),
        pltpu.SemaphoreType.DMA,
        pltpu.SemaphoreType.DMA,
        pltpu.SemaphoreType.DMA,
        pltpu.SemaphoreType.DMA,
    ],
    compiler_params=pltpu.CompilerParams(use_tc_tiling_on_sc=False),
)


def _prep_body(degT_ref, xt_ref, y_ref, dinv_ref):
    d = jnp.sum(degT_ref[...], axis=1, keepdims=True) + 1.0
    dinv = lax.rsqrt(d)
    dinv_ref[...] = dinv
    y_ref[...] = dinv * xt_ref[...]


def _prep(degT, xt):
    bn = 1000
    return pl.pallas_call(
        _prep_body,
        grid=(N // bn,),
        in_specs=[
            pl.BlockSpec((bn, NC), lambda i: (i, 0)),
            pl.BlockSpec((bn, F), lambda i: (i, 0)),
        ],
        out_specs=[
            pl.BlockSpec((bn, F), lambda i: (i, 0)),
            pl.BlockSpec((bn, 1), lambda i: (i, 0)),
        ],
        out_shape=[
            jax.ShapeDtypeStruct((N, F), jnp.float32),
            jax.ShapeDtypeStruct((N, 1), jnp.float32),
        ],
    )(degT, xt)


def _wfuse_body(wz, lz, bz, lbz, wr, lr, br, lbr, wh, lh, bh, lbh, att,
                wzp, wrp, whp, bzp, brp, bhp, probs):
    hp = jax.lax.Precision.HIGHEST
    wzp[...] = jnp.dot(wz[...], lz[0:C, :], precision=hp)
    wrp[...] = jnp.dot(wr[...], lr[0:C, :], precision=hp)
    whp[...] = jnp.dot(wh[...], lh[0:C, :], precision=hp)
    bzp[...] = jnp.dot(bz[...], lz[0:C, :], precision=hp) + lbz[...]
    brp[...] = jnp.dot(br[...], lr[0:C, :], precision=hp) + lbr[...]
    bhp[...] = jnp.dot(bh[...], lh[0:C, :], precision=hp) + lbh[...]
    a = att[...]
    e = jnp.exp(a - jnp.max(a))
    probs[...] = e / jnp.sum(e)


def _wfuse(wz, lz, bz, lbz, wr, lr, br, lbr, wh, lh, bh, lbh, att):
    f32 = jnp.float32
    return pl.pallas_call(
        _wfuse_body,
        out_shape=[
            jax.ShapeDtypeStruct((C, C), f32),
            jax.ShapeDtypeStruct((C, C), f32),
            jax.ShapeDtypeStruct((C, C), f32),
            jax.ShapeDtypeStruct((1, C), f32),
            jax.ShapeDtypeStruct((1, C), f32),
            jax.ShapeDtypeStruct((1, C), f32),
            jax.ShapeDtypeStruct((1, P), f32),
        ],
    )(wz, lz, bz, lbz, wr, lr, br, lbr, wh, lh, bh, lbh, att)


def _gru_body(agg_ref, dinv_ref, wzp_ref, wrp_ref, whp_ref, lz_ref, lr_ref,
              lh_ref, bzp_ref, brp_ref, bhp_ref, probs_ref, wl_ref, bl_ref,
              out_ref):
    s = agg_ref[...] * dinv_ref[...]
    wzp = wzp_ref[...]
    wrp = wrp_ref[...]
    whp = whp_ref[...]
    uz = lz_ref[C:2 * C, :]
    ur = lr_ref[C:2 * C, :]
    uh = lh_ref[C:2 * C, :]
    bzp = bzp_ref[...]
    brp = brp_ref[...]
    bhp = bhp_ref[...]
    probs = probs_ref[...]
    bn = s.shape[0]
    h = jnp.zeros((bn, C), jnp.float32)
    ha = jnp.zeros((bn, C), jnp.float32)
    for t in range(P):
        st = s[:, t * C:(t + 1) * C]
        z = jax.nn.sigmoid(jnp.dot(st, wzp) + jnp.dot(h, uz) + bzp)
        r = jax.nn.sigmoid(jnp.dot(st, wrp) + jnp.dot(h, ur) + brp)
        ht = jnp.tanh(jnp.dot(st, whp) + jnp.dot(h * r, uh) + bhp)
        h = z * h + (1.0 - z) * ht
        ha = ha + probs[0, t] * h
    out_ref[...] = jnp.dot(jnp.maximum(ha, 0.0), wl_ref[...]) + bl_ref[...]


def _gru(agg, dinv, wzp, wrp, whp, lz, lr, lh, bzp, brp, bhp, probs, wl, bl):
    bn = 1000
    full = lambda shape: pl.BlockSpec(shape, lambda i: tuple(0 for _ in shape))
    return pl.pallas_call(
        _gru_body,
        grid=(N // bn,),
        in_specs=[
            pl.BlockSpec((bn, F), lambda i: (i, 0)),
            pl.BlockSpec((bn, 1), lambda i: (i, 0)),
            full((C, C)), full((C, C)), full((C, C)),
            full((2 * C, C)), full((2 * C, C)), full((2 * C, C)),
            full((1, C)), full((1, C)), full((1, C)),
            full((1, P)), full((C, P)), full((1, P)),
        ],
        out_specs=pl.BlockSpec((bn, P), lambda i: (i, 0)),
        out_shape=jax.ShapeDtypeStruct((N, P), jnp.float32),
    )(agg, dinv, wzp, wrp, whp, lz, lr, lh, bzp, brp, bhp, probs, wl, bl)


@jax.jit
def kernel(x, edge_index, W_z, b_z, L_z, lb_z, W_r, b_r, L_r, lb_r,
           W_h, b_h, L_h, lb_h, att, W_lin, b_lin):
    src2d = edge_index[0].reshape(EROWS, EB)
    dst2d = edge_index[1].reshape(EROWS, EB)
    xt = jnp.transpose(x, (0, 2, 1)).reshape(N, F)

    deg_flat = _deg_call(dst2d)
    degT = deg_flat.reshape(NC, N).T
    y, dinv = _prep(degT, xt)
    y3 = y.reshape(N, F // CP, CP).transpose(1, 0, 2)
    agg3 = _agg_call(src2d, dst2d, y3)
    agg = agg3.transpose(1, 0, 2).reshape(N, F)
    wzp, wrp, whp, bzp, brp, bhp, probs = _wfuse(
        W_z, L_z, b_z.reshape(1, C), lb_z.reshape(1, C),
        W_r, L_r, b_r.reshape(1, C), lb_r.reshape(1, C),
        W_h, L_h, b_h.reshape(1, C), lb_h.reshape(1, C),
        att.reshape(1, P))
    return _gru(agg, dinv, wzp, wrp, whp, L_z, L_r, L_h, bzp, brp, bhp,
                probs, W_lin, b_lin.reshape(1, P))


# speedup vs baseline: 1.1314x; 1.1314x over previous
import functools
import jax
import jax.numpy as jnp
from jax import lax
from jax.experimental import pallas as pl
from jax.experimental.pallas import tpu as pltpu
from jax.experimental.pallas import tpu_sc as plsc

N = 10000
C = 128
P = 12
F = C * P
E = 320000
EROWS = 2560
EB = 125
NC = 2
NS = 16
ROWS_PER_TILE_DEG = EROWS // (NC * NS)
ROWS_PER_TILE_AGG = EROWS // NS
CP = 48
NSL = F // CP
NPASS = NSL // NC
NSLICE = N // 10

_sc_mesh = plsc.VectorSubcoreMesh(
    core_axis_name="c", subcore_axis_name="s", num_cores=NC, num_subcores=NS)


def _deg_body(dst_hbm, out_hbm, dst_v, ones_v, zeros_v, stage_v, acc_sh, sem):
    cid = lax.axis_index("c")
    sid = lax.axis_index("s")
    for i in range(63):
        zeros_v[pl.ds(i * 16, 16)] = jnp.zeros((16,), jnp.float32)
    for i in range(8):
        ones_v[pl.ds(i * 16, 16)] = jnp.ones((16,), jnp.float32)

    @pl.when(sid < 10)
    def _():
        pltpu.sync_copy(zeros_v.at[pl.ds(0, NSLICE)],
                        acc_sh.at[pl.ds(sid * NSLICE, NSLICE)])

    plsc.subcore_barrier()

    row0 = (cid * NS + sid) * ROWS_PER_TILE_DEG
    pltpu.sync_copy(dst_hbm.at[pl.ds(row0, ROWS_PER_TILE_DEG)], dst_v)

    def body(j, carry):
        pltpu.sync_copy(ones_v.at[pl.ds(0, EB)], acc_sh.at[dst_v.at[j]],
                        add=True)
        return carry

    lax.fori_loop(0, ROWS_PER_TILE_DEG, body, 0)
    plsc.subcore_barrier()

    @pl.when(sid < 10)
    def _():
        pltpu.sync_copy(acc_sh.at[pl.ds(sid * NSLICE, NSLICE)], stage_v)
        pltpu.sync_copy(stage_v,
                        out_hbm.at[pl.ds(cid * N + sid * NSLICE, NSLICE)])


_deg_call = pl.kernel(
    _deg_body,
    out_type=jax.ShapeDtypeStruct((NC * N,), jnp.float32),
    mesh=_sc_mesh,
    scratch_types=[
        pltpu.VMEM((ROWS_PER_TILE_DEG, EB), jnp.int32),
        pltpu.VMEM((128,), jnp.float32),
        pltpu.VMEM((1008,), jnp.float32),
        pltpu.VMEM((NSLICE,), jnp.float32),
        pltpu.VMEM_SHARED((N,), jnp.float32),
        pltpu.SemaphoreType.DMA,
    ],
)


NPAIR = ROWS_PER_TILE_AGG // 2
STROWS = 250


def _agg_body(src_hbm, dst_hbm, y_hbm, out_hbm, src_v, dst_v, buf0, buf1,
              st0, st1, table_sh, acc_sh, g0, g1, s0, s1):
    cid = lax.axis_index("c")
    sid = lax.axis_index("s")
    row0 = sid * ROWS_PER_TILE_AGG
    pltpu.sync_copy(src_hbm.at[pl.ds(row0, ROWS_PER_TILE_AGG)], src_v)
    pltpu.sync_copy(dst_hbm.at[pl.ds(row0, ROWS_PER_TILE_AGG)], dst_v)

    def one_pass(p, pcarry):
        sl = cid * NPASS + p

        @pl.when(sid < 10)
        def _():
            def stage_in(k, carry):
                r = sid * NSLICE + k * STROWS
                pltpu.sync_copy(y_hbm.at[sl, pl.ds(r, STROWS), :], st0)
                pltpu.sync_copy(st0, table_sh.at[pl.ds(r, STROWS), :])
                pltpu.sync_copy(st0, acc_sh.at[pl.ds(r, STROWS), :])
                return carry

            lax.fori_loop(0, NSLICE // STROWS, stage_in, 0)

        plsc.subcore_barrier()

        def body(i, carry):
            r = i * 2
            d0 = pltpu.async_copy(table_sh.at[src_v.at[r]], buf0, g0)
            d1 = pltpu.async_copy(table_sh.at[src_v.at[r + 1]], buf1, g1)
            d0.wait()
            pltpu.sync_copy(buf0, acc_sh.at[dst_v.at[r]], add=True)
            d1.wait()
            pltpu.sync_copy(buf1, acc_sh.at[dst_v.at[r + 1]], add=True)
            return carry

        lax.fori_loop(0, NPAIR, body, 0)
        plsc.subcore_barrier()

        @pl.when(sid < 10)
        def _():
            def stage_out(k, carry):
                r = sid * NSLICE + k * STROWS
                pltpu.sync_copy(acc_sh.at[pl.ds(r, STROWS), :], st0)
                pltpu.sync_copy(st0, out_hbm.at[sl, pl.ds(r, STROWS), :])
                return carry

            lax.fori_loop(0, NSLICE // STROWS, stage_out, 0)

        return pcarry

    lax.fori_loop(0, NPASS, one_pass, 0)


_agg_call = pl.kernel(
    _agg_body,
    out_type=jax.ShapeDtypeStruct((NSL, N, CP), jnp.float32),
    mesh=_sc_mesh,
    scratch_types=[
        pltpu.VMEM((ROWS_PER_TILE_AGG, EB), jnp.int32),
        pltpu.VMEM((ROWS_PER_TILE_AGG, EB), jnp.int32),
        pltpu.VMEM((EB, CP), jnp.float32),
        pltpu.VMEM((EB, CP), jnp.float32),
        pltpu.VMEM((STROWS, CP), jnp.float32),
        pltpu.VMEM((STROWS, CP), jnp.float32),
        pltpu.VMEM_SHARED((N, CP), jnp.float32),
        pltpu.VMEM_SHARED((N, CP), jnp.float32),
        pltpu.SemaphoreType.DMA,
        pltpu.SemaphoreType.DMA,
        pltpu.SemaphoreType.DMA,
        pltpu.SemaphoreType.DMA,
    ],
    compiler_params=pltpu.CompilerParams(use_tc_tiling_on_sc=False),
)


def _prep_body(deg_ref, xt_ref, y_ref, dinv_ref):
    d = jnp.sum(deg_ref[...], axis=1, keepdims=True) + 1.0
    dinv = lax.rsqrt(d)
    dinv_ref[...] = dinv
    xt = xt_ref[...]
    for s in range(NSL):
        y_ref[s] = dinv * xt[:, s * CP:(s + 1) * CP]


def _prep(deg, xt):
    bn = 1000
    return pl.pallas_call(
        _prep_body,
        grid=(N // bn,),
        in_specs=[
            pl.BlockSpec((bn, NC), lambda i: (i, 0)),
            pl.BlockSpec((bn, F), lambda i: (i, 0)),
        ],
        out_specs=[
            pl.BlockSpec((NSL, bn, CP), lambda i: (0, i, 0)),
            pl.BlockSpec((bn, 1), lambda i: (i, 0)),
        ],
        out_shape=[
            jax.ShapeDtypeStruct((NSL, N, CP), jnp.float32),
            jax.ShapeDtypeStruct((N, 1), jnp.float32),
        ],
    )(deg, xt)


def _wfuse_body(wz, lz, bz, lbz, wr, lr, br, lbr, wh, lh, bh, lbh, att,
                wzp, wrp, whp, bzp, brp, bhp, probs):
    hp = jax.lax.Precision.HIGHEST
    wzp[...] = jnp.dot(wz[...], lz[0:C, :], precision=hp)
    wrp[...] = jnp.dot(wr[...], lr[0:C, :], precision=hp)
    whp[...] = jnp.dot(wh[...], lh[0:C, :], precision=hp)
    bzp[...] = jnp.dot(bz[...], lz[0:C, :], precision=hp) + lbz[...]
    brp[...] = jnp.dot(br[...], lr[0:C, :], precision=hp) + lbr[...]
    bhp[...] = jnp.dot(bh[...], lh[0:C, :], precision=hp) + lbh[...]
    a = att[...]
    e = jnp.exp(a - jnp.max(a))
    probs[...] = e / jnp.sum(e)


def _wfuse(wz, lz, bz, lbz, wr, lr, br, lbr, wh, lh, bh, lbh, att):
    f32 = jnp.float32
    return pl.pallas_call(
        _wfuse_body,
        out_shape=[
            jax.ShapeDtypeStruct((C, C), f32),
            jax.ShapeDtypeStruct((C, C), f32),
            jax.ShapeDtypeStruct((C, C), f32),
            jax.ShapeDtypeStruct((1, C), f32),
            jax.ShapeDtypeStruct((1, C), f32),
            jax.ShapeDtypeStruct((1, C), f32),
            jax.ShapeDtypeStruct((1, P), f32),
        ],
    )(wz, lz, bz, lbz, wr, lr, br, lbr, wh, lh, bh, lbh, att)


def _gru_body(agg_ref, dinv_ref, wzp_ref, wrp_ref, whp_ref, lz_ref, lr_ref,
              lh_ref, bzp_ref, brp_ref, bhp_ref, probs_ref, wl_ref, bl_ref,
              out_ref):
    dinv = dinv_ref[...]
    wzp = wzp_ref[...]
    wrp = wrp_ref[...]
    whp = whp_ref[...]
    uz = lz_ref[C:2 * C, :]
    ur = lr_ref[C:2 * C, :]
    uh = lh_ref[C:2 * C, :]
    bzp = bzp_ref[...]
    brp = brp_ref[...]
    bhp = bhp_ref[...]
    probs = probs_ref[...]
    bn = dinv.shape[0]
    h = jnp.zeros((bn, C), jnp.float32)
    ha = jnp.zeros((bn, C), jnp.float32)
    for t in range(P):
        pieces = []
        f0, f1 = t * C, (t + 1) * C
        s = f0 // CP
        while f0 < f1:
            o = f0 % CP
            take = min(CP - o, f1 - f0)
            pieces.append(agg_ref[s][:, o:o + take])
            f0 += take
            s += 1
        st = dinv * jnp.concatenate(pieces, axis=1)
        z = jax.nn.sigmoid(jnp.dot(st, wzp) + jnp.dot(h, uz) + bzp)
        r = jax.nn.sigmoid(jnp.dot(st, wrp) + jnp.dot(h, ur) + brp)
        ht = jnp.tanh(jnp.dot(st, whp) + jnp.dot(h * r, uh) + bhp)
        h = z * h + (1.0 - z) * ht
        ha = ha + probs[0, t] * h
    out_ref[...] = jnp.dot(jnp.maximum(ha, 0.0), wl_ref[...]) + bl_ref[...]


def _gru(agg3, dinv, wzp, wrp, whp, lz, lr, lh, bzp, brp, bhp, probs, wl, bl):
    bn = 1000
    full = lambda shape: pl.BlockSpec(shape, lambda i: tuple(0 for _ in shape))
    return pl.pallas_call(
        _gru_body,
        grid=(N // bn,),
        in_specs=[
            pl.BlockSpec((NSL, bn, CP), lambda i: (0, i, 0)),
            pl.BlockSpec((bn, 1), lambda i: (i, 0)),
            full((C, C)), full((C, C)), full((C, C)),
            full((2 * C, C)), full((2 * C, C)), full((2 * C, C)),
            full((1, C)), full((1, C)), full((1, C)),
            full((1, P)), full((C, P)), full((1, P)),
        ],
        out_specs=pl.BlockSpec((bn, P), lambda i: (i, 0)),
        out_shape=jax.ShapeDtypeStruct((N, P), jnp.float32),
    )(agg3, dinv, wzp, wrp, whp, lz, lr, lh, bzp, brp, bhp, probs, wl, bl)


@jax.jit
def kernel(x, edge_index, W_z, b_z, L_z, lb_z, W_r, b_r, L_r, lb_r,
           W_h, b_h, L_h, lb_h, att, W_lin, b_lin):
    src2d = edge_index[0].reshape(EROWS, EB)
    dst2d = edge_index[1].reshape(EROWS, EB)
    xt = jnp.transpose(x, (0, 2, 1)).reshape(N, F)

    degT = _deg_call(dst2d).reshape(NC, N).T
    y3, dinv = _prep(degT, xt)
    agg3 = _agg_call(src2d, dst2d, y3)
    wzp, wrp, whp, bzp, brp, bhp, probs = _wfuse(
        W_z, L_z, b_z.reshape(1, C), lb_z.reshape(1, C),
        W_r, L_r, b_r.reshape(1, C), lb_r.reshape(1, C),
        W_h, L_h, b_h.reshape(1, C), lb_h.reshape(1, C),
        att.reshape(1, P))
    return _gru(agg3, dinv, wzp, wrp, whp, L_z, L_r, L_h, bzp, brp, bhp,
                probs, W_lin, b_lin.reshape(1, P))


# speedup vs baseline: 1.5824x; 1.3986x over previous
import functools
import jax
import jax.numpy as jnp
from jax import lax
from jax.experimental import pallas as pl
from jax.experimental.pallas import tpu as pltpu
from jax.experimental.pallas import tpu_sc as plsc

N = 10000
C = 128
P = 12
F = C * P
E = 320000
EROWS = 2560
EB = 125
NC = 2
NS = 16
ROWS_PER_TILE_DEG = EROWS // (NC * NS)
ROWS_PER_TILE_AGG = EROWS // NS
CP = 48
NSL = F // CP
NPASS = NSL // NC
NSLICE = N // 10

_sc_mesh = plsc.VectorSubcoreMesh(
    core_axis_name="c", subcore_axis_name="s", num_cores=NC, num_subcores=NS)


def _deg_body(dst_hbm, out_hbm, dst_v, ones_v, zeros_v, stage_v, acc_sh, sem):
    cid = lax.axis_index("c")
    sid = lax.axis_index("s")
    for i in range(63):
        zeros_v[pl.ds(i * 16, 16)] = jnp.zeros((16,), jnp.float32)
    for i in range(8):
        ones_v[pl.ds(i * 16, 16)] = jnp.ones((16,), jnp.float32)

    @pl.when(sid < 10)
    def _():
        pltpu.sync_copy(zeros_v.at[pl.ds(0, NSLICE)],
                        acc_sh.at[pl.ds(sid * NSLICE, NSLICE)])

    plsc.subcore_barrier()

    row0 = (cid * NS + sid) * ROWS_PER_TILE_DEG
    pltpu.sync_copy(dst_hbm.at[pl.ds(row0, ROWS_PER_TILE_DEG)], dst_v)

    def body(j, carry):
        pltpu.sync_copy(ones_v.at[pl.ds(0, EB)], acc_sh.at[dst_v.at[j]],
                        add=True)
        return carry

    lax.fori_loop(0, ROWS_PER_TILE_DEG, body, 0)
    plsc.subcore_barrier()

    @pl.when(sid < 10)
    def _():
        pltpu.sync_copy(acc_sh.at[pl.ds(sid * NSLICE, NSLICE)], stage_v)
        pltpu.sync_copy(stage_v,
                        out_hbm.at[pl.ds(cid * N + sid * NSLICE, NSLICE)])


_deg_call = pl.kernel(
    _deg_body,
    out_type=jax.ShapeDtypeStruct((NC * N,), jnp.float32),
    mesh=_sc_mesh,
    scratch_types=[
        pltpu.VMEM((ROWS_PER_TILE_DEG, EB), jnp.int32),
        pltpu.VMEM((128,), jnp.float32),
        pltpu.VMEM((1008,), jnp.float32),
        pltpu.VMEM((NSLICE,), jnp.float32),
        pltpu.VMEM_SHARED((N,), jnp.float32),
        pltpu.SemaphoreType.DMA,
    ],
)


NPAIR = ROWS_PER_TILE_AGG // 2
STROWS = 125


NB = ROWS_PER_TILE_AGG
NGRP = NB // 4


def _agg_body(src_hbm, dst_hbm, y_hbm, out_hbm, src_v, dst_v, buf0, buf1,
              buf2, buf3, st0, table_sh, acc_sh, g0, g1, g2, g3):
    cid = lax.axis_index("c")
    sid = lax.axis_index("s")
    row0 = sid * ROWS_PER_TILE_AGG
    pltpu.sync_copy(src_hbm.at[pl.ds(row0, ROWS_PER_TILE_AGG)], src_v)
    pltpu.sync_copy(dst_hbm.at[pl.ds(row0, ROWS_PER_TILE_AGG)], dst_v)

    def one_pass(p, pcarry):
        sl = cid * NPASS + p

        @pl.when(sid < 10)
        def _():
            def stage_in(k, carry):
                r = sid * NSLICE + k * STROWS
                pltpu.sync_copy(y_hbm.at[sl, pl.ds(r, STROWS), :], st0)
                pltpu.sync_copy(st0, table_sh.at[pl.ds(r, STROWS), :])
                pltpu.sync_copy(st0, acc_sh.at[pl.ds(r, STROWS), :])
                return carry

            lax.fori_loop(0, NSLICE // STROWS, stage_in, 0)

        plsc.subcore_barrier()

        pltpu.async_copy(table_sh.at[src_v.at[0]], buf0, g0)
        pltpu.async_copy(table_sh.at[src_v.at[1]], buf1, g1)

        def body(i, carry):
            r = i * 4
            pltpu.make_async_copy(table_sh.at[src_v.at[r]], buf0, g0).wait()
            pltpu.make_async_copy(table_sh.at[src_v.at[r + 1]], buf1,
                                  g1).wait()
            d2 = pltpu.async_copy(table_sh.at[src_v.at[r + 2]], buf2, g2)
            d3 = pltpu.async_copy(table_sh.at[src_v.at[r + 3]], buf3, g3)
            pltpu.sync_copy(buf0, acc_sh.at[dst_v.at[r]], add=True)
            pltpu.sync_copy(buf1, acc_sh.at[dst_v.at[r + 1]], add=True)
            d2.wait()
            d3.wait()
            pltpu.async_copy(table_sh.at[src_v.at[r + 4]], buf0, g0)
            pltpu.async_copy(table_sh.at[src_v.at[r + 5]], buf1, g1)
            pltpu.sync_copy(buf2, acc_sh.at[dst_v.at[r + 2]], add=True)
            pltpu.sync_copy(buf3, acc_sh.at[dst_v.at[r + 3]], add=True)
            return carry

        lax.fori_loop(0, NGRP - 1, body, 0)

        re = NB - 4
        pltpu.make_async_copy(table_sh.at[src_v.at[re]], buf0, g0).wait()
        pltpu.make_async_copy(table_sh.at[src_v.at[re + 1]], buf1, g1).wait()
        d2 = pltpu.async_copy(table_sh.at[src_v.at[re + 2]], buf2, g2)
        d3 = pltpu.async_copy(table_sh.at[src_v.at[re + 3]], buf3, g3)
        pltpu.sync_copy(buf0, acc_sh.at[dst_v.at[re]], add=True)
        pltpu.sync_copy(buf1, acc_sh.at[dst_v.at[re + 1]], add=True)
        d2.wait()
        d3.wait()
        pltpu.sync_copy(buf2, acc_sh.at[dst_v.at[re + 2]], add=True)
        pltpu.sync_copy(buf3, acc_sh.at[dst_v.at[re + 3]], add=True)
        plsc.subcore_barrier()

        @pl.when(sid < 10)
        def _():
            def stage_out(k, carry):
                r = sid * NSLICE + k * STROWS
                pltpu.sync_copy(acc_sh.at[pl.ds(r, STROWS), :], st0)
                pltpu.sync_copy(st0, out_hbm.at[sl, pl.ds(r, STROWS), :])
                return carry

            lax.fori_loop(0, NSLICE // STROWS, stage_out, 0)

        return pcarry

    lax.fori_loop(0, NPASS, one_pass, 0)


_agg_call = pl.kernel(
    _agg_body,
    out_type=jax.ShapeDtypeStruct((NSL, N, CP), jnp.float32),
    mesh=_sc_mesh,
    scratch_types=[
        pltpu.VMEM((ROWS_PER_TILE_AGG, EB), jnp.int32),
        pltpu.VMEM((ROWS_PER_TILE_AGG, EB), jnp.int32),
        pltpu.VMEM((EB, CP), jnp.float32),
        pltpu.VMEM((EB, CP), jnp.float32),
        pltpu.VMEM((EB, CP), jnp.float32),
        pltpu.VMEM((EB, CP), jnp.float32),
        pltpu.VMEM((STROWS, CP), jnp.float32),
        pltpu.VMEM_SHARED((N, CP), jnp.float32),
        pltpu.VMEM_SHARED((N, CP), jnp.float32),
        pltpu.SemaphoreType.DMA,
        pltpu.SemaphoreType.DMA,
        pltpu.SemaphoreType.DMA,
        pltpu.SemaphoreType.DMA,
    ],
    compiler_params=pltpu.CompilerParams(use_tc_tiling_on_sc=False),
)


def _prep_body(deg_ref, xt_ref, y_ref, dinv_ref):
    d = jnp.sum(deg_ref[...], axis=1, keepdims=True) + 1.0
    dinv = lax.rsqrt(d)
    dinv_ref[...] = dinv
    xt = xt_ref[...]
    for s in range(NSL):
        y_ref[s] = dinv * xt[:, s * CP:(s + 1) * CP]


def _prep(deg, xt):
    bn = 1000
    return pl.pallas_call(
        _prep_body,
        grid=(N // bn,),
        in_specs=[
            pl.BlockSpec((bn, NC), lambda i: (i, 0)),
            pl.BlockSpec((bn, F), lambda i: (i, 0)),
        ],
        out_specs=[
            pl.BlockSpec((NSL, bn, CP), lambda i: (0, i, 0)),
            pl.BlockSpec((bn, 1), lambda i: (i, 0)),
        ],
        out_shape=[
            jax.ShapeDtypeStruct((NSL, N, CP), jnp.float32),
            jax.ShapeDtypeStruct((N, 1), jnp.float32),
        ],
    )(deg, xt)


def _wfuse_body(wz, lz, bz, lbz, wr, lr, br, lbr, wh, lh, bh, lbh, att,
                wzp, wrp, whp, bzp, brp, bhp, probs):
    hp = jax.lax.Precision.HIGHEST
    wzp[...] = jnp.dot(wz[...], lz[0:C, :], precision=hp)
    wrp[...] = jnp.dot(wr[...], lr[0:C, :], precision=hp)
    whp[...] = jnp.dot(wh[...], lh[0:C, :], precision=hp)
    bzp[...] = jnp.dot(bz[...], lz[0:C, :], precision=hp) + lbz[...]
    brp[...] = jnp.dot(br[...], lr[0:C, :], precision=hp) + lbr[...]
    bhp[...] = jnp.dot(bh[...], lh[0:C, :], precision=hp) + lbh[...]
    a = att[...]
    e = jnp.exp(a - jnp.max(a))
    probs[...] = e / jnp.sum(e)


def _wfuse(wz, lz, bz, lbz, wr, lr, br, lbr, wh, lh, bh, lbh, att):
    f32 = jnp.float32
    return pl.pallas_call(
        _wfuse_body,
        out_shape=[
            jax.ShapeDtypeStruct((C, C), f32),
            jax.ShapeDtypeStruct((C, C), f32),
            jax.ShapeDtypeStruct((C, C), f32),
            jax.ShapeDtypeStruct((1, C), f32),
            jax.ShapeDtypeStruct((1, C), f32),
            jax.ShapeDtypeStruct((1, C), f32),
            jax.ShapeDtypeStruct((1, P), f32),
        ],
    )(wz, lz, bz, lbz, wr, lr, br, lbr, wh, lh, bh, lbh, att)


def _gru_body(agg_ref, dinv_ref, wzp_ref, wrp_ref, whp_ref, lz_ref, lr_ref,
              lh_ref, bzp_ref, brp_ref, bhp_ref, probs_ref, wl_ref, bl_ref,
              out_ref):
    dinv = dinv_ref[...]
    wzp = wzp_ref[...]
    wrp = wrp_ref[...]
    whp = whp_ref[...]
    uz = lz_ref[C:2 * C, :]
    ur = lr_ref[C:2 * C, :]
    uh = lh_ref[C:2 * C, :]
    bzp = bzp_ref[...]
    brp = brp_ref[...]
    bhp = bhp_ref[...]
    probs = probs_ref[...]
    bn = dinv.shape[0]
    h = jnp.zeros((bn, C), jnp.float32)
    ha = jnp.zeros((bn, C), jnp.float32)
    for t in range(P):
        pieces = []
        f0, f1 = t * C, (t + 1) * C
        s = f0 // CP
        while f0 < f1:
            o = f0 % CP
            take = min(CP - o, f1 - f0)
            pieces.append(agg_ref[s][:, o:o + take])
            f0 += take
            s += 1
        st = dinv * jnp.concatenate(pieces, axis=1)
        z = jax.nn.sigmoid(jnp.dot(st, wzp) + jnp.dot(h, uz) + bzp)
        r = jax.nn.sigmoid(jnp.dot(st, wrp) + jnp.dot(h, ur) + brp)
        ht = jnp.tanh(jnp.dot(st, whp) + jnp.dot(h * r, uh) + bhp)
        h = z * h + (1.0 - z) * ht
        ha = ha + probs[0, t] * h
    out_ref[...] = jnp.dot(jnp.maximum(ha, 0.0), wl_ref[...]) + bl_ref[...]


def _gru(agg3, dinv, wzp, wrp, whp, lz, lr, lh, bzp, brp, bhp, probs, wl, bl):
    bn = 1000
    full = lambda shape: pl.BlockSpec(shape, lambda i: tuple(0 for _ in shape))
    return pl.pallas_call(
        _gru_body,
        grid=(N // bn,),
        in_specs=[
            pl.BlockSpec((NSL, bn, CP), lambda i: (0, i, 0)),
            pl.BlockSpec((bn, 1), lambda i: (i, 0)),
            full((C, C)), full((C, C)), full((C, C)),
            full((2 * C, C)), full((2 * C, C)), full((2 * C, C)),
            full((1, C)), full((1, C)), full((1, C)),
            full((1, P)), full((C, P)), full((1, P)),
        ],
        out_specs=pl.BlockSpec((bn, P), lambda i: (i, 0)),
        out_shape=jax.ShapeDtypeStruct((N, P), jnp.float32),
    )(agg3, dinv, wzp, wrp, whp, lz, lr, lh, bzp, brp, bhp, probs, wl, bl)


@jax.jit
def kernel(x, edge_index, W_z, b_z, L_z, lb_z, W_r, b_r, L_r, lb_r,
           W_h, b_h, L_h, lb_h, att, W_lin, b_lin):
    src2d = edge_index[0].reshape(EROWS, EB)
    dst2d = edge_index[1].reshape(EROWS, EB)
    xt = jnp.transpose(x, (0, 2, 1)).reshape(N, F)

    degT = _deg_call(dst2d).reshape(NC, N).T
    y3, dinv = _prep(degT, xt)
    agg3 = _agg_call(src2d, dst2d, y3)
    wzp, wrp, whp, bzp, brp, bhp, probs = _wfuse(
        W_z, L_z, b_z.reshape(1, C), lb_z.reshape(1, C),
        W_r, L_r, b_r.reshape(1, C), lb_r.reshape(1, C),
        W_h, L_h, b_h.reshape(1, C), lb_h.reshape(1, C),
        att.reshape(1, P))
    return _gru(agg3, dinv, wzp, wrp, whp, L_z, L_r, L_h, bzp, brp, bhp,
                probs, W_lin, b_lin.reshape(1, P))


# speedup vs baseline: 1.6167x; 1.0217x over previous
import functools
import jax
import jax.numpy as jnp
from jax import lax
from jax.experimental import pallas as pl
from jax.experimental.pallas import tpu as pltpu
from jax.experimental.pallas import tpu_sc as plsc

N = 10000
C = 128
P = 12
F = C * P
E = 320000
EROWS = 2560
EB = 125
NC = 2
NS = 16
ROWS_PER_TILE_DEG = EROWS // (NC * NS)
ROWS_PER_TILE_AGG = EROWS // NS
CP = 48
NSL = F // CP
NPASS = NSL // NC
NSLICE = N // 10

_sc_mesh = plsc.VectorSubcoreMesh(
    core_axis_name="c", subcore_axis_name="s", num_cores=NC, num_subcores=NS)


def _deg_body(dst_hbm, out_hbm, dst_v, ones_v, zeros_v, stage_v, acc_sh, sem):
    cid = lax.axis_index("c")
    sid = lax.axis_index("s")
    for i in range(63):
        zeros_v[pl.ds(i * 16, 16)] = jnp.zeros((16,), jnp.float32)
    for i in range(8):
        ones_v[pl.ds(i * 16, 16)] = jnp.ones((16,), jnp.float32)

    @pl.when(sid < 10)
    def _():
        pltpu.sync_copy(zeros_v.at[pl.ds(0, NSLICE)],
                        acc_sh.at[pl.ds(sid * NSLICE, NSLICE)])

    plsc.subcore_barrier()

    row0 = (cid * NS + sid) * ROWS_PER_TILE_DEG
    pltpu.sync_copy(dst_hbm.at[pl.ds(row0, ROWS_PER_TILE_DEG)], dst_v)

    def body(j, carry):
        pltpu.sync_copy(ones_v.at[pl.ds(0, EB)], acc_sh.at[dst_v.at[j]],
                        add=True)
        return carry

    lax.fori_loop(0, ROWS_PER_TILE_DEG, body, 0)
    plsc.subcore_barrier()

    @pl.when(sid < 10)
    def _():
        pltpu.sync_copy(acc_sh.at[pl.ds(sid * NSLICE, NSLICE)], stage_v)
        pltpu.sync_copy(stage_v,
                        out_hbm.at[pl.ds(cid * N + sid * NSLICE, NSLICE)])


_deg_call = pl.kernel(
    _deg_body,
    out_type=jax.ShapeDtypeStruct((NC * N,), jnp.float32),
    mesh=_sc_mesh,
    scratch_types=[
        pltpu.VMEM((ROWS_PER_TILE_DEG, EB), jnp.int32),
        pltpu.VMEM((128,), jnp.float32),
        pltpu.VMEM((1008,), jnp.float32),
        pltpu.VMEM((NSLICE,), jnp.float32),
        pltpu.VMEM_SHARED((N,), jnp.float32),
        pltpu.SemaphoreType.DMA,
    ],
)


NPAIR = ROWS_PER_TILE_AGG // 2
STROWS = 125


NB = ROWS_PER_TILE_AGG
NGRP = NB // 4


def _agg_body(base, half, src_hbm, dst_hbm, y_hbm, out_hbm, src_v, dst_v,
              buf0, buf1, buf2, buf3, st0, table_sh, acc_sh, g0, g1, g2, g3):
    cid = lax.axis_index("c")
    sid = lax.axis_index("s")
    row0 = sid * ROWS_PER_TILE_AGG
    pltpu.sync_copy(src_hbm.at[pl.ds(row0, ROWS_PER_TILE_AGG)], src_v)
    pltpu.sync_copy(dst_hbm.at[pl.ds(row0, ROWS_PER_TILE_AGG)], dst_v)

    def one_pass(p, pcarry):
        sl = cid * half + p

        @pl.when(sid < 10)
        def _():
            def stage_in(k, carry):
                r = sid * NSLICE + k * STROWS
                pltpu.sync_copy(y_hbm.at[base + sl, pl.ds(r, STROWS), :], st0)
                pltpu.sync_copy(st0, table_sh.at[pl.ds(r, STROWS), :])
                pltpu.sync_copy(st0, acc_sh.at[pl.ds(r, STROWS), :])
                return carry

            lax.fori_loop(0, NSLICE // STROWS, stage_in, 0)

        plsc.subcore_barrier()

        pltpu.async_copy(table_sh.at[src_v.at[0]], buf0, g0)
        pltpu.async_copy(table_sh.at[src_v.at[1]], buf1, g1)

        def body(i, carry):
            r = i * 4
            pltpu.make_async_copy(table_sh.at[src_v.at[r]], buf0, g0).wait()
            pltpu.make_async_copy(table_sh.at[src_v.at[r + 1]], buf1,
                                  g1).wait()
            d2 = pltpu.async_copy(table_sh.at[src_v.at[r + 2]], buf2, g2)
            d3 = pltpu.async_copy(table_sh.at[src_v.at[r + 3]], buf3, g3)
            pltpu.sync_copy(buf0, acc_sh.at[dst_v.at[r]], add=True)
            pltpu.sync_copy(buf1, acc_sh.at[dst_v.at[r + 1]], add=True)
            d2.wait()
            d3.wait()
            pltpu.async_copy(table_sh.at[src_v.at[r + 4]], buf0, g0)
            pltpu.async_copy(table_sh.at[src_v.at[r + 5]], buf1, g1)
            pltpu.sync_copy(buf2, acc_sh.at[dst_v.at[r + 2]], add=True)
            pltpu.sync_copy(buf3, acc_sh.at[dst_v.at[r + 3]], add=True)
            return carry

        lax.fori_loop(0, NGRP - 1, body, 0)

        re = NB - 4
        pltpu.make_async_copy(table_sh.at[src_v.at[re]], buf0, g0).wait()
        pltpu.make_async_copy(table_sh.at[src_v.at[re + 1]], buf1, g1).wait()
        d2 = pltpu.async_copy(table_sh.at[src_v.at[re + 2]], buf2, g2)
        d3 = pltpu.async_copy(table_sh.at[src_v.at[re + 3]], buf3, g3)
        pltpu.sync_copy(buf0, acc_sh.at[dst_v.at[re]], add=True)
        pltpu.sync_copy(buf1, acc_sh.at[dst_v.at[re + 1]], add=True)
        d2.wait()
        d3.wait()
        pltpu.sync_copy(buf2, acc_sh.at[dst_v.at[re + 2]], add=True)
        pltpu.sync_copy(buf3, acc_sh.at[dst_v.at[re + 3]], add=True)
        plsc.subcore_barrier()

        @pl.when(sid < 10)
        def _():
            def stage_out(k, carry):
                r = sid * NSLICE + k * STROWS
                pltpu.sync_copy(acc_sh.at[pl.ds(r, STROWS), :], st0)
                pltpu.sync_copy(st0, out_hbm.at[sl, pl.ds(r, STROWS), :])
                return carry

            lax.fori_loop(0, NSLICE // STROWS, stage_out, 0)

        return pcarry

    lax.fori_loop(0, half, one_pass, 0)


HALF = NPASS // 2


def _make_agg_call(base):
    return pl.kernel(
        functools.partial(_agg_body, base, HALF),
        out_type=jax.ShapeDtypeStruct((NSL // 2, N, CP), jnp.float32),
        mesh=_sc_mesh,
        scratch_types=[
        pltpu.VMEM((ROWS_PER_TILE_AGG, EB), jnp.int32),
        pltpu.VMEM((ROWS_PER_TILE_AGG, EB), jnp.int32),
        pltpu.VMEM((EB, CP), jnp.float32),
        pltpu.VMEM((EB, CP), jnp.float32),
        pltpu.VMEM((EB, CP), jnp.float32),
        pltpu.VMEM((EB, CP), jnp.float32),
            pltpu.VMEM((STROWS, CP), jnp.float32),
            pltpu.VMEM_SHARED((N, CP), jnp.float32),
            pltpu.VMEM_SHARED((N, CP), jnp.float32),
            pltpu.SemaphoreType.DMA,
            pltpu.SemaphoreType.DMA,
            pltpu.SemaphoreType.DMA,
            pltpu.SemaphoreType.DMA,
        ],
        compiler_params=pltpu.CompilerParams(use_tc_tiling_on_sc=False),
    )


_agg_call_a = _make_agg_call(0)
_agg_call_b = _make_agg_call(NSL // 2)


def _prep_body(deg_ref, xt_ref, y_ref, dinv_ref):
    d = jnp.sum(deg_ref[...], axis=1, keepdims=True) + 1.0
    dinv = lax.rsqrt(d)
    dinv_ref[...] = dinv
    xt = xt_ref[...]
    for s in range(NSL):
        y_ref[s] = dinv * xt[:, s * CP:(s + 1) * CP]


def _prep(deg, xt):
    bn = 1000
    return pl.pallas_call(
        _prep_body,
        grid=(N // bn,),
        in_specs=[
            pl.BlockSpec((bn, NC), lambda i: (i, 0)),
            pl.BlockSpec((bn, F), lambda i: (i, 0)),
        ],
        out_specs=[
            pl.BlockSpec((NSL, bn, CP), lambda i: (0, i, 0)),
            pl.BlockSpec((bn, 1), lambda i: (i, 0)),
        ],
        out_shape=[
            jax.ShapeDtypeStruct((NSL, N, CP), jnp.float32),
            jax.ShapeDtypeStruct((N, 1), jnp.float32),
        ],
    )(deg, xt)


def _wfuse_body(wz, lz, bz, lbz, wr, lr, br, lbr, wh, lh, bh, lbh, att,
                wzp, wrp, whp, bzp, brp, bhp, probs):
    hp = jax.lax.Precision.HIGHEST
    wzp[...] = jnp.dot(wz[...], lz[0:C, :], precision=hp)
    wrp[...] = jnp.dot(wr[...], lr[0:C, :], precision=hp)
    whp[...] = jnp.dot(wh[...], lh[0:C, :], precision=hp)
    bzp[...] = jnp.dot(bz[...], lz[0:C, :], precision=hp) + lbz[...]
    brp[...] = jnp.dot(br[...], lr[0:C, :], precision=hp) + lbr[...]
    bhp[...] = jnp.dot(bh[...], lh[0:C, :], precision=hp) + lbh[...]
    a = att[...]
    e = jnp.exp(a - jnp.max(a))
    probs[...] = e / jnp.sum(e)


def _wfuse(wz, lz, bz, lbz, wr, lr, br, lbr, wh, lh, bh, lbh, att):
    f32 = jnp.float32
    return pl.pallas_call(
        _wfuse_body,
        out_shape=[
            jax.ShapeDtypeStruct((C, C), f32),
            jax.ShapeDtypeStruct((C, C), f32),
            jax.ShapeDtypeStruct((C, C), f32),
            jax.ShapeDtypeStruct((1, C), f32),
            jax.ShapeDtypeStruct((1, C), f32),
            jax.ShapeDtypeStruct((1, C), f32),
            jax.ShapeDtypeStruct((1, P), f32),
        ],
    )(wz, lz, bz, lbz, wr, lr, br, lbr, wh, lh, bh, lbh, att)


PH = P // 2


def _gru_steps(agg_ref, dinv, h, ha, wzp, wrp, whp, uz, ur, uh, bzp, brp,
               bhp, probs, t0):
    for tt in range(PH):
        pieces = []
        f0, f1 = tt * C, (tt + 1) * C
        s = f0 // CP
        while f0 < f1:
            o = f0 % CP
            take = min(CP - o, f1 - f0)
            pieces.append(agg_ref[s][:, o:o + take])
            f0 += take
            s += 1
        st = dinv * jnp.concatenate(pieces, axis=1)
        z = jax.nn.sigmoid(jnp.dot(st, wzp) + jnp.dot(h, uz) + bzp)
        r = jax.nn.sigmoid(jnp.dot(st, wrp) + jnp.dot(h, ur) + brp)
        ht = jnp.tanh(jnp.dot(st, whp) + jnp.dot(h * r, uh) + bhp)
        h = z * h + (1.0 - z) * ht
        ha = ha + probs[0, t0 + tt] * h
    return h, ha


def _gru_a_body(agg_ref, dinv_ref, wzp_ref, wrp_ref, whp_ref, lz_ref, lr_ref,
                lh_ref, bzp_ref, brp_ref, bhp_ref, probs_ref, h_ref, ha_ref):
    dinv = dinv_ref[...]
    bn = dinv.shape[0]
    h = jnp.zeros((bn, C), jnp.float32)
    ha = jnp.zeros((bn, C), jnp.float32)
    h, ha = _gru_steps(agg_ref, dinv, h, ha, wzp_ref[...], wrp_ref[...],
                       whp_ref[...], lz_ref[C:2 * C, :], lr_ref[C:2 * C, :],
                       lh_ref[C:2 * C, :], bzp_ref[...], brp_ref[...],
                       bhp_ref[...], probs_ref[...], 0)
    h_ref[...] = h
    ha_ref[...] = ha


def _gru_b_body(agg_ref, dinv_ref, wzp_ref, wrp_ref, whp_ref, lz_ref, lr_ref,
                lh_ref, bzp_ref, brp_ref, bhp_ref, probs_ref, h_ref, ha_ref,
                wl_ref, bl_ref, out_ref):
    dinv = dinv_ref[...]
    h, ha = _gru_steps(agg_ref, dinv, h_ref[...], ha_ref[...], wzp_ref[...],
                       wrp_ref[...], whp_ref[...], lz_ref[C:2 * C, :],
                       lr_ref[C:2 * C, :], lh_ref[C:2 * C, :], bzp_ref[...],
                       brp_ref[...], bhp_ref[...], probs_ref[...], PH)
    out_ref[...] = jnp.dot(jnp.maximum(ha, 0.0), wl_ref[...]) + bl_ref[...]


_BN = 1000
_full = lambda shape: pl.BlockSpec(shape, lambda i: tuple(0 for _ in shape))
_WSPECS = [
    _full((C, C)), _full((C, C)), _full((C, C)),
    _full((2 * C, C)), _full((2 * C, C)), _full((2 * C, C)),
    _full((1, C)), _full((1, C)), _full((1, C)),
    _full((1, P)),
]
_AGGSPEC = pl.BlockSpec((NSL // 2, _BN, CP), lambda i: (0, i, 0))
_NSPEC = lambda w: pl.BlockSpec((_BN, w), lambda i: (i, 0))


def _gru_a(agg_a, dinv, *weights):
    return pl.pallas_call(
        _gru_a_body,
        grid=(N // _BN,),
        in_specs=[_AGGSPEC, _NSPEC(1)] + _WSPECS,
        out_specs=[_NSPEC(C), _NSPEC(C)],
        out_shape=[jax.ShapeDtypeStruct((N, C), jnp.float32),
                   jax.ShapeDtypeStruct((N, C), jnp.float32)],
    )(agg_a, dinv, *weights)


def _gru_b(agg_b, dinv, h, ha, wl, bl, *weights):
    return pl.pallas_call(
        _gru_b_body,
        grid=(N // _BN,),
        in_specs=[_AGGSPEC, _NSPEC(1)] + _WSPECS
        + [_NSPEC(C), _NSPEC(C), _full((C, P)), _full((1, P))],
        out_specs=_NSPEC(P),
        out_shape=jax.ShapeDtypeStruct((N, P), jnp.float32),
    )(agg_b, dinv, *weights, h, ha, wl, bl)


@jax.jit
def kernel(x, edge_index, W_z, b_z, L_z, lb_z, W_r, b_r, L_r, lb_r,
           W_h, b_h, L_h, lb_h, att, W_lin, b_lin):
    src2d = edge_index[0].reshape(EROWS, EB)
    dst2d = edge_index[1].reshape(EROWS, EB)
    xt = jnp.transpose(x, (0, 2, 1)).reshape(N, F)

    degT = _deg_call(dst2d).reshape(NC, N).T
    y3, dinv = _prep(degT, xt)
    wzp, wrp, whp, bzp, brp, bhp, probs = _wfuse(
        W_z, L_z, b_z.reshape(1, C), lb_z.reshape(1, C),
        W_r, L_r, b_r.reshape(1, C), lb_r.reshape(1, C),
        W_h, L_h, b_h.reshape(1, C), lb_h.reshape(1, C),
        att.reshape(1, P))
    weights = (wzp, wrp, whp, L_z, L_r, L_h, bzp, brp, bhp, probs)
    agg_a = _agg_call_a(src2d, dst2d, y3)
    agg_b = _agg_call_b(src2d, dst2d, y3)
    h, ha = _gru_a(agg_a, dinv, *weights)
    return _gru_b(agg_b, dinv, h, ha, W_lin, b_lin.reshape(1, P), *weights)


# speedup vs baseline: 1.6349x; 1.0112x over previous
import functools
import jax
import jax.numpy as jnp
from jax import lax
from jax.experimental import pallas as pl
from jax.experimental.pallas import tpu as pltpu
from jax.experimental.pallas import tpu_sc as plsc

N = 10000
C = 128
P = 12
F = C * P
E = 320000
EROWS = 2560
EB = 125
NC = 2
NS = 16
ROWS_PER_TILE_DEG = EROWS // (NC * NS)
ROWS_PER_TILE_AGG = EROWS // NS
CP = 48
NSL = F // CP
NPASS = NSL // NC
NSLICE = N // 10

_sc_mesh = plsc.VectorSubcoreMesh(
    core_axis_name="c", subcore_axis_name="s", num_cores=NC, num_subcores=NS)


def _deg_body(dst_hbm, out_hbm, dst_v, ones_v, zeros_v, stage_v, acc_sh, sem):
    cid = lax.axis_index("c")
    sid = lax.axis_index("s")
    for i in range(63):
        zeros_v[pl.ds(i * 16, 16)] = jnp.zeros((16,), jnp.float32)
    for i in range(8):
        ones_v[pl.ds(i * 16, 16)] = jnp.ones((16,), jnp.float32)

    @pl.when(sid < 10)
    def _():
        pltpu.sync_copy(zeros_v.at[pl.ds(0, NSLICE)],
                        acc_sh.at[pl.ds(sid * NSLICE, NSLICE)])

    plsc.subcore_barrier()

    row0 = (cid * NS + sid) * ROWS_PER_TILE_DEG
    pltpu.sync_copy(dst_hbm.at[pl.ds(row0, ROWS_PER_TILE_DEG)], dst_v)

    def body(j, carry):
        pltpu.sync_copy(ones_v.at[pl.ds(0, EB)], acc_sh.at[dst_v.at[j]],
                        add=True)
        return carry

    lax.fori_loop(0, ROWS_PER_TILE_DEG, body, 0)
    plsc.subcore_barrier()

    @pl.when(sid < 10)
    def _():
        pltpu.sync_copy(acc_sh.at[pl.ds(sid * NSLICE, NSLICE)], stage_v)
        pltpu.sync_copy(stage_v,
                        out_hbm.at[pl.ds(cid * N + sid * NSLICE, NSLICE)])


_deg_call = pl.kernel(
    _deg_body,
    out_type=jax.ShapeDtypeStruct((NC * N,), jnp.float32),
    mesh=_sc_mesh,
    scratch_types=[
        pltpu.VMEM((ROWS_PER_TILE_DEG, EB), jnp.int32),
        pltpu.VMEM((128,), jnp.float32),
        pltpu.VMEM((1008,), jnp.float32),
        pltpu.VMEM((NSLICE,), jnp.float32),
        pltpu.VMEM_SHARED((N,), jnp.float32),
        pltpu.SemaphoreType.DMA,
    ],
)


NPAIR = ROWS_PER_TILE_AGG // 2
STROWS = 125


NB = ROWS_PER_TILE_AGG
NGRP = NB // 4


def _agg_body(base, half, src_hbm, dst_hbm, y_hbm, out_hbm, src_v, dst_v,
              buf0, buf1, buf2, buf3, st0, table_sh, acc_sh, g0, g1, g2, g3):
    cid = lax.axis_index("c")
    sid = lax.axis_index("s")
    row0 = sid * ROWS_PER_TILE_AGG
    pltpu.sync_copy(src_hbm.at[pl.ds(row0, ROWS_PER_TILE_AGG)], src_v)
    pltpu.sync_copy(dst_hbm.at[pl.ds(row0, ROWS_PER_TILE_AGG)], dst_v)

    def one_pass(p, pcarry):
        sl = cid * half + p

        @pl.when(sid < 10)
        def _():
            def stage_in(k, carry):
                r = sid * NSLICE + k * STROWS
                pltpu.sync_copy(y_hbm.at[base + sl, pl.ds(r, STROWS), :], st0)
                pltpu.sync_copy(st0, table_sh.at[pl.ds(r, STROWS), :])
                pltpu.sync_copy(st0, acc_sh.at[pl.ds(r, STROWS), :])
                return carry

            lax.fori_loop(0, NSLICE // STROWS, stage_in, 0)

        plsc.subcore_barrier()

        pltpu.async_copy(table_sh.at[src_v.at[0]], buf0, g0)
        pltpu.async_copy(table_sh.at[src_v.at[1]], buf1, g1)

        def body(i, carry):
            r = i * 4
            pltpu.make_async_copy(table_sh.at[src_v.at[r]], buf0, g0).wait()
            pltpu.make_async_copy(table_sh.at[src_v.at[r + 1]], buf1,
                                  g1).wait()
            d2 = pltpu.async_copy(table_sh.at[src_v.at[r + 2]], buf2, g2)
            d3 = pltpu.async_copy(table_sh.at[src_v.at[r + 3]], buf3, g3)
            pltpu.sync_copy(buf0, acc_sh.at[dst_v.at[r]], add=True)
            pltpu.sync_copy(buf1, acc_sh.at[dst_v.at[r + 1]], add=True)
            d2.wait()
            d3.wait()
            pltpu.async_copy(table_sh.at[src_v.at[r + 4]], buf0, g0)
            pltpu.async_copy(table_sh.at[src_v.at[r + 5]], buf1, g1)
            pltpu.sync_copy(buf2, acc_sh.at[dst_v.at[r + 2]], add=True)
            pltpu.sync_copy(buf3, acc_sh.at[dst_v.at[r + 3]], add=True)
            return carry

        lax.fori_loop(0, NGRP - 1, body, 0)

        re = NB - 4
        pltpu.make_async_copy(table_sh.at[src_v.at[re]], buf0, g0).wait()
        pltpu.make_async_copy(table_sh.at[src_v.at[re + 1]], buf1, g1).wait()
        d2 = pltpu.async_copy(table_sh.at[src_v.at[re + 2]], buf2, g2)
        d3 = pltpu.async_copy(table_sh.at[src_v.at[re + 3]], buf3, g3)
        pltpu.sync_copy(buf0, acc_sh.at[dst_v.at[re]], add=True)
        pltpu.sync_copy(buf1, acc_sh.at[dst_v.at[re + 1]], add=True)
        d2.wait()
        d3.wait()
        pltpu.sync_copy(buf2, acc_sh.at[dst_v.at[re + 2]], add=True)
        pltpu.sync_copy(buf3, acc_sh.at[dst_v.at[re + 3]], add=True)
        plsc.subcore_barrier()

        @pl.when(sid < 10)
        def _():
            def stage_out(k, carry):
                r = sid * NSLICE + k * STROWS
                pltpu.sync_copy(acc_sh.at[pl.ds(r, STROWS), :], st0)
                pltpu.sync_copy(st0, out_hbm.at[sl, pl.ds(r, STROWS), :])
                return carry

            lax.fori_loop(0, NSLICE // STROWS, stage_out, 0)

        return pcarry

    lax.fori_loop(0, half, one_pass, 0)


HALF = NPASS // 2


def _make_agg_call(base):
    return pl.kernel(
        functools.partial(_agg_body, base, HALF),
        out_type=jax.ShapeDtypeStruct((NSL // 2, N, CP), jnp.float32),
        mesh=_sc_mesh,
        scratch_types=[
        pltpu.VMEM((ROWS_PER_TILE_AGG, EB), jnp.int32),
        pltpu.VMEM((ROWS_PER_TILE_AGG, EB), jnp.int32),
        pltpu.VMEM((EB, CP), jnp.float32),
        pltpu.VMEM((EB, CP), jnp.float32),
        pltpu.VMEM((EB, CP), jnp.float32),
        pltpu.VMEM((EB, CP), jnp.float32),
            pltpu.VMEM((STROWS, CP), jnp.float32),
            pltpu.VMEM_SHARED((N, CP), jnp.float32),
            pltpu.VMEM_SHARED((N, CP), jnp.float32),
            pltpu.SemaphoreType.DMA,
            pltpu.SemaphoreType.DMA,
            pltpu.SemaphoreType.DMA,
            pltpu.SemaphoreType.DMA,
        ],
        compiler_params=pltpu.CompilerParams(use_tc_tiling_on_sc=False),
    )


_agg_call_a = _make_agg_call(0)
_agg_call_b = _make_agg_call(NSL // 2)


def _prep_body(deg_ref, xt_ref, y_ref, dinv_ref):
    d = jnp.sum(deg_ref[...], axis=1, keepdims=True) + 1.0
    dinv = lax.rsqrt(d)
    dinv_ref[...] = dinv
    xt = xt_ref[...]
    for s in range(NSL):
        y_ref[s] = dinv * xt[:, s * CP:(s + 1) * CP]


def _prep(deg, xt):
    bn = 1000
    return pl.pallas_call(
        _prep_body,
        grid=(N // bn,),
        in_specs=[
            pl.BlockSpec((bn, NC), lambda i: (i, 0)),
            pl.BlockSpec((bn, F), lambda i: (i, 0)),
        ],
        out_specs=[
            pl.BlockSpec((NSL, bn, CP), lambda i: (0, i, 0)),
            pl.BlockSpec((bn, 1), lambda i: (i, 0)),
        ],
        out_shape=[
            jax.ShapeDtypeStruct((NSL, N, CP), jnp.float32),
            jax.ShapeDtypeStruct((N, 1), jnp.float32),
        ],
    )(deg, xt)


def _wfuse_body(wz, lz, bz, lbz, wr, lr, br, lbr, wh, lh, bh, lbh, att,
                wzp, wrp, whp, bzp, brp, bhp, probs):
    hp = jax.lax.Precision.HIGHEST
    wzp[...] = jnp.dot(wz[...], lz[0:C, :], precision=hp)
    wrp[...] = jnp.dot(wr[...], lr[0:C, :], precision=hp)
    whp[...] = jnp.dot(wh[...], lh[0:C, :], precision=hp)
    bzp[...] = jnp.dot(bz[...], lz[0:C, :], precision=hp) + lbz[...]
    brp[...] = jnp.dot(br[...], lr[0:C, :], precision=hp) + lbr[...]
    bhp[...] = jnp.dot(bh[...], lh[0:C, :], precision=hp) + lbh[...]
    a = att[...]
    e = jnp.exp(a - jnp.max(a))
    probs[...] = e / jnp.sum(e)


def _wfuse(wz, lz, bz, lbz, wr, lr, br, lbr, wh, lh, bh, lbh, att):
    f32 = jnp.float32
    return pl.pallas_call(
        _wfuse_body,
        out_shape=[
            jax.ShapeDtypeStruct((C, C), f32),
            jax.ShapeDtypeStruct((C, C), f32),
            jax.ShapeDtypeStruct((C, C), f32),
            jax.ShapeDtypeStruct((1, C), f32),
            jax.ShapeDtypeStruct((1, C), f32),
            jax.ShapeDtypeStruct((1, C), f32),
            jax.ShapeDtypeStruct((1, P), f32),
        ],
    )(wz, lz, bz, lbz, wr, lr, br, lbr, wh, lh, bh, lbh, att)


PH = P // 2


def _gru_steps(agg_ref, dinv, h, ha, wzp, wrp, whp, uz, ur, uh, bzp, brp,
               bhp, probs, t0):
    for tt in range(PH):
        pieces = []
        f0, f1 = tt * C, (tt + 1) * C
        s = f0 // CP
        while f0 < f1:
            o = f0 % CP
            take = min(CP - o, f1 - f0)
            pieces.append(agg_ref[s][:, o:o + take])
            f0 += take
            s += 1
        st = dinv * jnp.concatenate(pieces, axis=1)
        z = jax.nn.sigmoid(jnp.dot(st, wzp) + jnp.dot(h, uz) + bzp)
        r = jax.nn.sigmoid(jnp.dot(st, wrp) + jnp.dot(h, ur) + brp)
        ht = jnp.tanh(jnp.dot(st, whp) + jnp.dot(h * r, uh) + bhp)
        h = z * h + (1.0 - z) * ht
        ha = ha + probs[0, t0 + tt] * h
    return h, ha


def _gru_a_body(agg_ref, dinv_ref, wzp_ref, wrp_ref, whp_ref, lz_ref, lr_ref,
                lh_ref, bzp_ref, brp_ref, bhp_ref, probs_ref, h_ref, ha_ref):
    dinv = dinv_ref[...]
    bn = dinv.shape[0]
    h = jnp.zeros((bn, C), jnp.float32)
    ha = jnp.zeros((bn, C), jnp.float32)
    h, ha = _gru_steps(agg_ref, dinv, h, ha, wzp_ref[...], wrp_ref[...],
                       whp_ref[...], lz_ref[C:2 * C, :], lr_ref[C:2 * C, :],
                       lh_ref[C:2 * C, :], bzp_ref[...], brp_ref[...],
                       bhp_ref[...], probs_ref[...], 0)
    h_ref[...] = h
    ha_ref[...] = ha


def _gru_b_body(agg_ref, dinv_ref, wzp_ref, wrp_ref, whp_ref, lz_ref, lr_ref,
                lh_ref, bzp_ref, brp_ref, bhp_ref, probs_ref, h_ref, ha_ref,
                wl_ref, bl_ref, out_ref):
    dinv = dinv_ref[...]
    h, ha = _gru_steps(agg_ref, dinv, h_ref[...], ha_ref[...], wzp_ref[...],
                       wrp_ref[...], whp_ref[...], lz_ref[C:2 * C, :],
                       lr_ref[C:2 * C, :], lh_ref[C:2 * C, :], bzp_ref[...],
                       brp_ref[...], bhp_ref[...], probs_ref[...], PH)
    out_ref[...] = jnp.dot(jnp.maximum(ha, 0.0), wl_ref[...]) + bl_ref[...]


_BN = 2000
_full = lambda shape: pl.BlockSpec(shape, lambda i: tuple(0 for _ in shape))
_WSPECS = [
    _full((C, C)), _full((C, C)), _full((C, C)),
    _full((2 * C, C)), _full((2 * C, C)), _full((2 * C, C)),
    _full((1, C)), _full((1, C)), _full((1, C)),
    _full((1, P)),
]
_AGGSPEC = pl.BlockSpec((NSL // 2, _BN, CP), lambda i: (0, i, 0))
_NSPEC = lambda w: pl.BlockSpec((_BN, w), lambda i: (i, 0))


def _gru_a(agg_a, dinv, *weights):
    return pl.pallas_call(
        _gru_a_body,
        grid=(N // _BN,),
        in_specs=[_AGGSPEC, _NSPEC(1)] + _WSPECS,
        out_specs=[_NSPEC(C), _NSPEC(C)],
        out_shape=[jax.ShapeDtypeStruct((N, C), jnp.float32),
                   jax.ShapeDtypeStruct((N, C), jnp.float32)],
    )(agg_a, dinv, *weights)


def _gru_b(agg_b, dinv, h, ha, wl, bl, *weights):
    return pl.pallas_call(
        _gru_b_body,
        grid=(N // _BN,),
        in_specs=[_AGGSPEC, _NSPEC(1)] + _WSPECS
        + [_NSPEC(C), _NSPEC(C), _full((C, P)), _full((1, P))],
        out_specs=_NSPEC(P),
        out_shape=jax.ShapeDtypeStruct((N, P), jnp.float32),
    )(agg_b, dinv, *weights, h, ha, wl, bl)


@jax.jit
def kernel(x, edge_index, W_z, b_z, L_z, lb_z, W_r, b_r, L_r, lb_r,
           W_h, b_h, L_h, lb_h, att, W_lin, b_lin):
    src2d = edge_index[0].reshape(EROWS, EB)
    dst2d = edge_index[1].reshape(EROWS, EB)
    xt = jnp.transpose(x, (0, 2, 1)).reshape(N, F)

    degT = _deg_call(dst2d).reshape(NC, N).T
    y3, dinv = _prep(degT, xt)
    wzp, wrp, whp, bzp, brp, bhp, probs = _wfuse(
        W_z, L_z, b_z.reshape(1, C), lb_z.reshape(1, C),
        W_r, L_r, b_r.reshape(1, C), lb_r.reshape(1, C),
        W_h, L_h, b_h.reshape(1, C), lb_h.reshape(1, C),
        att.reshape(1, P))
    weights = (wzp, wrp, whp, L_z, L_r, L_h, bzp, brp, bhp, probs)
    agg_a = _agg_call_a(src2d, dst2d, y3)
    agg_b = _agg_call_b(src2d, dst2d, y3)
    h, ha = _gru_a(agg_a, dinv, *weights)
    return _gru_b(agg_b, dinv, h, ha, W_lin, b_lin.reshape(1, P), *weights)


# speedup vs baseline: 1.6353x; 1.0003x over previous
import functools
import jax
import jax.numpy as jnp
from jax import lax
from jax.experimental import pallas as pl
from jax.experimental.pallas import tpu as pltpu
from jax.experimental.pallas import tpu_sc as plsc

N = 10000
C = 128
P = 12
F = C * P
E = 320000
EROWS = 2560
EB = 125
NC = 2
NS = 16
ROWS_PER_TILE_DEG = EROWS // (NC * NS)
ROWS_PER_TILE_AGG = EROWS // NS
CP = 48
NSL = F // CP
NPASS = NSL // NC
NSLICE = N // 10

_sc_mesh = plsc.VectorSubcoreMesh(
    core_axis_name="c", subcore_axis_name="s", num_cores=NC, num_subcores=NS)


def _deg_body(dst_hbm, out_hbm, dst_v, ones_v, zeros_v, stage_v, acc_sh, sem):
    cid = lax.axis_index("c")
    sid = lax.axis_index("s")
    for i in range(63):
        zeros_v[pl.ds(i * 16, 16)] = jnp.zeros((16,), jnp.float32)
    for i in range(8):
        ones_v[pl.ds(i * 16, 16)] = jnp.ones((16,), jnp.float32)

    @pl.when(sid < 10)
    def _():
        pltpu.sync_copy(zeros_v.at[pl.ds(0, NSLICE)],
                        acc_sh.at[pl.ds(sid * NSLICE, NSLICE)])

    plsc.subcore_barrier()

    row0 = (cid * NS + sid) * ROWS_PER_TILE_DEG
    pltpu.sync_copy(dst_hbm.at[pl.ds(row0, ROWS_PER_TILE_DEG)], dst_v)

    def body(j, carry):
        pltpu.sync_copy(ones_v.at[pl.ds(0, EB)], acc_sh.at[dst_v.at[j]],
                        add=True)
        return carry

    lax.fori_loop(0, ROWS_PER_TILE_DEG, body, 0)
    plsc.subcore_barrier()

    @pl.when(sid < 10)
    def _():
        pltpu.sync_copy(acc_sh.at[pl.ds(sid * NSLICE, NSLICE)], stage_v)
        pltpu.sync_copy(stage_v,
                        out_hbm.at[pl.ds(cid * N + sid * NSLICE, NSLICE)])


_deg_call = pl.kernel(
    _deg_body,
    out_type=jax.ShapeDtypeStruct((NC * N,), jnp.float32),
    mesh=_sc_mesh,
    scratch_types=[
        pltpu.VMEM((ROWS_PER_TILE_DEG, EB), jnp.int32),
        pltpu.VMEM((128,), jnp.float32),
        pltpu.VMEM((1008,), jnp.float32),
        pltpu.VMEM((NSLICE,), jnp.float32),
        pltpu.VMEM_SHARED((N,), jnp.float32),
        pltpu.SemaphoreType.DMA,
    ],
)


STROWS = 125


NB = ROWS_PER_TILE_AGG
NGRP = NB // 4


def _agg_body(base, half, src_hbm, dst_hbm, y_hbm, out_hbm, src_v, dst_v,
              buf0, buf1, buf2, buf3, st0, table_sh, acc_sh, g0, g1, g2, g3):
    cid = lax.axis_index("c")
    sid = lax.axis_index("s")
    row0 = sid * ROWS_PER_TILE_AGG
    pltpu.sync_copy(src_hbm.at[pl.ds(row0, ROWS_PER_TILE_AGG)], src_v)
    pltpu.sync_copy(dst_hbm.at[pl.ds(row0, ROWS_PER_TILE_AGG)], dst_v)

    def one_pass(p, pcarry):
        sl = cid * half + p

        @pl.when(sid < 10)
        def _():
            def stage_in(k, carry):
                r = sid * NSLICE + k * STROWS
                pltpu.sync_copy(y_hbm.at[base + sl, pl.ds(r, STROWS), :], st0)
                pltpu.sync_copy(st0, table_sh.at[pl.ds(r, STROWS), :])
                pltpu.sync_copy(st0, acc_sh.at[pl.ds(r, STROWS), :])
                return carry

            lax.fori_loop(0, NSLICE // STROWS, stage_in, 0)

        plsc.subcore_barrier()

        pltpu.async_copy(table_sh.at[src_v.at[0]], buf0, g0)
        pltpu.async_copy(table_sh.at[src_v.at[1]], buf1, g1)

        def body(i, carry):
            r = i * 4
            pltpu.make_async_copy(table_sh.at[src_v.at[r]], buf0, g0).wait()
            pltpu.make_async_copy(table_sh.at[src_v.at[r + 1]], buf1,
                                  g1).wait()
            d2 = pltpu.async_copy(table_sh.at[src_v.at[r + 2]], buf2, g2)
            d3 = pltpu.async_copy(table_sh.at[src_v.at[r + 3]], buf3, g3)
            pltpu.sync_copy(buf0, acc_sh.at[dst_v.at[r]], add=True)
            pltpu.sync_copy(buf1, acc_sh.at[dst_v.at[r + 1]], add=True)
            d2.wait()
            d3.wait()
            pltpu.async_copy(table_sh.at[src_v.at[r + 4]], buf0, g0)
            pltpu.async_copy(table_sh.at[src_v.at[r + 5]], buf1, g1)
            pltpu.sync_copy(buf2, acc_sh.at[dst_v.at[r + 2]], add=True)
            pltpu.sync_copy(buf3, acc_sh.at[dst_v.at[r + 3]], add=True)
            return carry

        lax.fori_loop(0, NGRP - 1, body, 0)

        re = NB - 4
        pltpu.make_async_copy(table_sh.at[src_v.at[re]], buf0, g0).wait()
        pltpu.make_async_copy(table_sh.at[src_v.at[re + 1]], buf1, g1).wait()
        d2 = pltpu.async_copy(table_sh.at[src_v.at[re + 2]], buf2, g2)
        d3 = pltpu.async_copy(table_sh.at[src_v.at[re + 3]], buf3, g3)
        pltpu.sync_copy(buf0, acc_sh.at[dst_v.at[re]], add=True)
        pltpu.sync_copy(buf1, acc_sh.at[dst_v.at[re + 1]], add=True)
        d2.wait()
        d3.wait()
        pltpu.sync_copy(buf2, acc_sh.at[dst_v.at[re + 2]], add=True)
        pltpu.sync_copy(buf3, acc_sh.at[dst_v.at[re + 3]], add=True)
        plsc.subcore_barrier()

        @pl.when(sid < 10)
        def _():
            def stage_out(k, carry):
                r = sid * NSLICE + k * STROWS
                pltpu.sync_copy(acc_sh.at[pl.ds(r, STROWS), :], st0)
                pltpu.sync_copy(st0, out_hbm.at[sl, pl.ds(r, STROWS), :])
                return carry

            lax.fori_loop(0, NSLICE // STROWS, stage_out, 0)

        return pcarry

    lax.fori_loop(0, half, one_pass, 0)


HALF = NPASS // 2


def _make_agg_call(base):
    return pl.kernel(
        functools.partial(_agg_body, base, HALF),
        out_type=jax.ShapeDtypeStruct((NSL // 2, N, CP), jnp.float32),
        mesh=_sc_mesh,
        scratch_types=[
        pltpu.VMEM((ROWS_PER_TILE_AGG, EB), jnp.int32),
        pltpu.VMEM((ROWS_PER_TILE_AGG, EB), jnp.int32),
        pltpu.VMEM((EB, CP), jnp.float32),
        pltpu.VMEM((EB, CP), jnp.float32),
        pltpu.VMEM((EB, CP), jnp.float32),
        pltpu.VMEM((EB, CP), jnp.float32),
            pltpu.VMEM((STROWS, CP), jnp.float32),
            pltpu.VMEM_SHARED((N, CP), jnp.float32),
            pltpu.VMEM_SHARED((N, CP), jnp.float32),
            pltpu.SemaphoreType.DMA,
            pltpu.SemaphoreType.DMA,
            pltpu.SemaphoreType.DMA,
            pltpu.SemaphoreType.DMA,
        ],
        compiler_params=pltpu.CompilerParams(use_tc_tiling_on_sc=False),
    )


_agg_call_a = _make_agg_call(0)
_agg_call_b = _make_agg_call(NSL // 2)


def _prep_body(deg_ref, xt_ref, y_ref, dinv_ref):
    d = jnp.sum(deg_ref[...], axis=1, keepdims=True) + 1.0
    dinv = lax.rsqrt(d)
    dinv_ref[...] = dinv
    xt = xt_ref[...]
    for s in range(NSL):
        y_ref[s] = dinv * xt[:, s * CP:(s + 1) * CP]


def _prep(deg, xt):
    bn = 1000
    return pl.pallas_call(
        _prep_body,
        grid=(N // bn,),
        in_specs=[
            pl.BlockSpec((bn, NC), lambda i: (i, 0)),
            pl.BlockSpec((bn, F), lambda i: (i, 0)),
        ],
        out_specs=[
            pl.BlockSpec((NSL, bn, CP), lambda i: (0, i, 0)),
            pl.BlockSpec((bn, 1), lambda i: (i, 0)),
        ],
        out_shape=[
            jax.ShapeDtypeStruct((NSL, N, CP), jnp.float32),
            jax.ShapeDtypeStruct((N, 1), jnp.float32),
        ],
    )(deg, xt)


def _wfuse_body(wz, lz, bz, lbz, wr, lr, br, lbr, wh, lh, bh, lbh, att,
                wzp, wrp, whp, bzp, brp, bhp, probs):
    hp = jax.lax.Precision.HIGHEST
    wzp[...] = jnp.dot(wz[...], lz[0:C, :], precision=hp)
    wrp[...] = jnp.dot(wr[...], lr[0:C, :], precision=hp)
    whp[...] = jnp.dot(wh[...], lh[0:C, :], precision=hp)
    bzp[...] = jnp.dot(bz[...], lz[0:C, :], precision=hp) + lbz[...]
    brp[...] = jnp.dot(br[...], lr[0:C, :], precision=hp) + lbr[...]
    bhp[...] = jnp.dot(bh[...], lh[0:C, :], precision=hp) + lbh[...]
    a = att[...]
    e = jnp.exp(a - jnp.max(a))
    probs[...] = e / jnp.sum(e)


def _wfuse(wz, lz, bz, lbz, wr, lr, br, lbr, wh, lh, bh, lbh, att):
    f32 = jnp.float32
    return pl.pallas_call(
        _wfuse_body,
        out_shape=[
            jax.ShapeDtypeStruct((C, C), f32),
            jax.ShapeDtypeStruct((C, C), f32),
            jax.ShapeDtypeStruct((C, C), f32),
            jax.ShapeDtypeStruct((1, C), f32),
            jax.ShapeDtypeStruct((1, C), f32),
            jax.ShapeDtypeStruct((1, C), f32),
            jax.ShapeDtypeStruct((1, P), f32),
        ],
    )(wz, lz, bz, lbz, wr, lr, br, lbr, wh, lh, bh, lbh, att)


PH = P // 2


def _gru_steps(agg_ref, dinv, h, ha, wzp, wrp, whp, uz, ur, uh, bzp, brp,
               bhp, probs, t0):
    for tt in range(PH):
        pieces = []
        f0, f1 = tt * C, (tt + 1) * C
        s = f0 // CP
        while f0 < f1:
            o = f0 % CP
            take = min(CP - o, f1 - f0)
            pieces.append(agg_ref[s][:, o:o + take])
            f0 += take
            s += 1
        st = dinv * jnp.concatenate(pieces, axis=1)
        z = jax.nn.sigmoid(jnp.dot(st, wzp) + jnp.dot(h, uz) + bzp)
        r = jax.nn.sigmoid(jnp.dot(st, wrp) + jnp.dot(h, ur) + brp)
        ht = jnp.tanh(jnp.dot(st, whp) + jnp.dot(h * r, uh) + bhp)
        h = z * h + (1.0 - z) * ht
        ha = ha + probs[0, t0 + tt] * h
    return h, ha


def _gru_a_body(agg_ref, dinv_ref, wzp_ref, wrp_ref, whp_ref, lz_ref, lr_ref,
                lh_ref, bzp_ref, brp_ref, bhp_ref, probs_ref, h_ref, ha_ref):
    dinv = dinv_ref[...]
    bn = dinv.shape[0]
    h = jnp.zeros((bn, C), jnp.float32)
    ha = jnp.zeros((bn, C), jnp.float32)
    h, ha = _gru_steps(agg_ref, dinv, h, ha, wzp_ref[...], wrp_ref[...],
                       whp_ref[...], lz_ref[C:2 * C, :], lr_ref[C:2 * C, :],
                       lh_ref[C:2 * C, :], bzp_ref[...], brp_ref[...],
                       bhp_ref[...], probs_ref[...], 0)
    h_ref[...] = h
    ha_ref[...] = ha


def _gru_b_body(agg_ref, dinv_ref, wzp_ref, wrp_ref, whp_ref, lz_ref, lr_ref,
                lh_ref, bzp_ref, brp_ref, bhp_ref, probs_ref, h_ref, ha_ref,
                wl_ref, bl_ref, out_ref):
    dinv = dinv_ref[...]
    h, ha = _gru_steps(agg_ref, dinv, h_ref[...], ha_ref[...], wzp_ref[...],
                       wrp_ref[...], whp_ref[...], lz_ref[C:2 * C, :],
                       lr_ref[C:2 * C, :], lh_ref[C:2 * C, :], bzp_ref[...],
                       brp_ref[...], bhp_ref[...], probs_ref[...], PH)
    out_ref[...] = jnp.dot(jnp.maximum(ha, 0.0), wl_ref[...]) + bl_ref[...]


_BN = 2000
_full = lambda shape: pl.BlockSpec(shape, lambda i: tuple(0 for _ in shape))
_WSPECS = [
    _full((C, C)), _full((C, C)), _full((C, C)),
    _full((2 * C, C)), _full((2 * C, C)), _full((2 * C, C)),
    _full((1, C)), _full((1, C)), _full((1, C)),
    _full((1, P)),
]
_AGGSPEC = pl.BlockSpec((NSL // 2, _BN, CP), lambda i: (0, i, 0))
_NSPEC = lambda w: pl.BlockSpec((_BN, w), lambda i: (i, 0))


def _gru_a(agg_a, dinv, *weights):
    return pl.pallas_call(
        _gru_a_body,
        grid=(N // _BN,),
        in_specs=[_AGGSPEC, _NSPEC(1)] + _WSPECS,
        out_specs=[_NSPEC(C), _NSPEC(C)],
        out_shape=[jax.ShapeDtypeStruct((N, C), jnp.float32),
                   jax.ShapeDtypeStruct((N, C), jnp.float32)],
    )(agg_a, dinv, *weights)


def _gru_b(agg_b, dinv, h, ha, wl, bl, *weights):
    return pl.pallas_call(
        _gru_b_body,
        grid=(N // _BN,),
        in_specs=[_AGGSPEC, _NSPEC(1)] + _WSPECS
        + [_NSPEC(C), _NSPEC(C), _full((C, P)), _full((1, P))],
        out_specs=_NSPEC(P),
        out_shape=jax.ShapeDtypeStruct((N, P), jnp.float32),
    )(agg_b, dinv, *weights, h, ha, wl, bl)


@jax.jit
def kernel(x, edge_index, W_z, b_z, L_z, lb_z, W_r, b_r, L_r, lb_r,
           W_h, b_h, L_h, lb_h, att, W_lin, b_lin):
    src2d = edge_index[0].reshape(EROWS, EB)
    dst2d = edge_index[1].reshape(EROWS, EB)
    xt = jnp.transpose(x, (0, 2, 1)).reshape(N, F)

    degT = _deg_call(dst2d).reshape(NC, N).T
    y3, dinv = _prep(degT, xt)
    wzp, wrp, whp, bzp, brp, bhp, probs = _wfuse(
        W_z, L_z, b_z.reshape(1, C), lb_z.reshape(1, C),
        W_r, L_r, b_r.reshape(1, C), lb_r.reshape(1, C),
        W_h, L_h, b_h.reshape(1, C), lb_h.reshape(1, C),
        att.reshape(1, P))
    weights = (wzp, wrp, whp, L_z, L_r, L_h, bzp, brp, bhp, probs)
    agg_a = _agg_call_a(src2d, dst2d, y3)
    agg_b = _agg_call_b(src2d, dst2d, y3)
    h, ha = _gru_a(agg_a, dinv, *weights)
    return _gru_b(agg_b, dinv, h, ha, W_lin, b_lin.reshape(1, P), *weights)
